# baseline (speedup 1.0000x reference)
import jax
import jax.numpy as jnp
from jax import lax
from jax.experimental import pallas as pl
from jax.experimental.pallas import tpu as pltpu
from jax.experimental.pallas import tpu_sc as plsc

NUM_ITEMS = 9000
NUM_USERS = 1000
N = NUM_ITEMS + NUM_USERS
E = 160000
IN_NODE = 128
H = 4
OUT = 32
HO = H * OUT

NT = 16
CHUNK = 80
CHUNKS_PER_TILE = 128
EDGES_PER_TILE = CHUNK * CHUNKS_PER_TILE
E_PAD = EDGES_PER_TILE * NT
NROWS = 10240
WROWS = NROWS // 32
U_PAD = 1024


def _tc_body(x_ref, wn_ref, fl_ref, fr_ref, fe_ref, we_ref,
             h_ref, elr_ref, c_ref):
    xb = x_ref[...]
    wn = wn_ref[...]
    hb = jnp.dot(xb, wn, preferred_element_type=jnp.float32)
    h_ref[...] = hb

    r16 = lax.broadcasted_iota(jnp.int32, (IN_NODE, 8), 0)
    c16 = lax.broadcasted_iota(jnp.int32, (IN_NODE, 8), 1)
    ma = jnp.where((r16 // OUT == c16) & (c16 < H), 1.0, 0.0).astype(jnp.float32)
    mb = jnp.where((r16 // OUT == c16 - H) & (c16 >= H), 1.0, 0.0).astype(jnp.float32)
    tl = hb * fl_ref[...]
    tr = hb * fr_ref[...]
    elr_ref[...] = (jnp.dot(tl, ma, preferred_element_type=jnp.float32)
                    + jnp.dot(tr, mb, preferred_element_type=jnp.float32))

    @pl.when(pl.program_id(0) == 0)
    def _():
        rc = lax.broadcasted_iota(jnp.int32, (IN_NODE, IN_NODE), 0)
        cc = lax.broadcasted_iota(jnp.int32, (IN_NODE, IN_NODE), 1)
        mc = jnp.where((rc // OUT == cc) & (cc < H), 1.0, 0.0).astype(jnp.float32)
        colsum = jnp.sum(we_ref[...], axis=0, keepdims=True)
        ce = colsum * fe_ref[...]
        c_ref[...] = jnp.dot(ce, mc, preferred_element_type=jnp.float32)


def _tc_project(x, wn, fl, fr, fe, we):
    br = 1000
    grid = N // br
    return pl.pallas_call(
        _tc_body,
        grid=(grid,),
        in_specs=[
            pl.BlockSpec((br, IN_NODE), lambda i: (i, 0)),
            pl.BlockSpec((IN_NODE, HO), lambda i: (0, 0)),
            pl.BlockSpec((1, HO), lambda i: (0, 0)),
            pl.BlockSpec((1, HO), lambda i: (0, 0)),
            pl.BlockSpec((1, HO), lambda i: (0, 0)),
            pl.BlockSpec((16, HO), lambda i: (0, 0)),
        ],
        out_specs=[
            pl.BlockSpec((br, IN_NODE), lambda i: (i, 0)),
            pl.BlockSpec((br, 8), lambda i: (i, 0)),
            pl.BlockSpec((1, IN_NODE), lambda i: (0, 0)),
        ],
        out_shape=[
            jax.ShapeDtypeStruct((N, IN_NODE), jnp.float32),
            jax.ShapeDtypeStruct((N, 8), jnp.float32),
            jax.ShapeDtypeStruct((1, IN_NODE), jnp.float32),
        ],
    )(x, wn, fl, fr, fe, we)


def _sc_weights(elr_hbm, src_hbm, dst_hbm, rwd_hbm, c_hbm, w_hbm,
                elr_v, c_v, src0, src1, dst0, dst1, rwd0, rwd1, wc0, wc1,
                semi0, semi1, semo0, semo1):
    wid = lax.axis_index("s") * 2 + lax.axis_index("c")
    lanes = lax.iota(jnp.int32, 16)
    srcb, dstb, rwdb, wc = [src0, src1], [dst0, dst1], [rwd0, rwd1], [wc0, wc1]
    semi, semo = [semi0, semi1], [semo0, semo1]

    pltpu.sync_copy(elr_hbm, elr_v)
    pltpu.sync_copy(c_hbm, c_v)
    cvec = c_v[pl.ds(0, 16)]
    c_sc = [cvec[hh] for hh in range(H)]

    ept = EDGES_PER_TILE // 2
    cpt = CHUNKS_PER_TILE // 2
    tile_base = wid * ept

    def _issue_idx(base, p):
        pltpu.async_copy(src_hbm.at[pl.ds(base, CHUNK)], srcb[p], semi[p])
        pltpu.async_copy(dst_hbm.at[pl.ds(base, CHUNK)], dstb[p], semi[p])
        pltpu.async_copy(rwd_hbm.at[pl.ds(base, CHUNK)], rwdb[p], semi[p])

    def _wait_idx(p):
        pltpu.make_async_copy(src_hbm.at[pl.ds(0, CHUNK)], srcb[p], semi[p]).wait()
        pltpu.make_async_copy(dst_hbm.at[pl.ds(0, CHUNK)], dstb[p], semi[p]).wait()
        pltpu.make_async_copy(rwd_hbm.at[pl.ds(0, CHUNK)], rwdb[p], semi[p]).wait()

    def _drain_wout(p):
        pltpu.make_async_copy(wc[p], w_hbm.at[pl.ds(0, CHUNK * 4)], semo[p]).wait()

    _issue_idx(tile_base, 0)
    _issue_idx(tile_base + CHUNK, 1)

    def _pair(i, carry):
        for p in range(2):
            c = 2 * i + p
            _wait_idx(p)

            @pl.when(i > 0)
            def _():
                _drain_wout(p)

            for g in range(CHUNK // 16):
                sv = srcb[p][pl.ds(g * 16, 16)]
                dv = dstb[p][pl.ds(g * 16, 16)]
                rv = rwdb[p][pl.ds(g * 16, 16)]
                e_vec = lanes + (g * 16)
                for hh in range(H):
                    eli = plsc.load_gather(elr_v, [sv * 8 + hh])
                    eri = plsc.load_gather(elr_v, [dv * 8 + (4 + hh)])
                    z = eli + eri + rv * c_sc[hh]
                    z = jnp.where(z > 0.0, z, z * 0.2)
                    w = jnp.exp(z)
                    plsc.store_scatter(wc[p], [e_vec * 4 + hh], w)

            base = tile_base + c * CHUNK
            pltpu.async_copy(wc[p], w_hbm.at[pl.ds(base * 4, CHUNK * 4)], semo[p])
            nbase = tile_base + jnp.minimum(c + 2, cpt - 1) * CHUNK
            _issue_idx(nbase, p)
        return carry

    lax.fori_loop(0, cpt // 2, _pair, 0)
    _drain_wout(0)
    _drain_wout(1)
    _wait_idx(0)
    _wait_idx(1)


def _run_sc_weights(elr_flat, srcp, dstp, rwdp, c16):
    mesh = plsc.VectorSubcoreMesh(core_axis_name="c", subcore_axis_name="s")
    fn = pl.kernel(
        _sc_weights, mesh=mesh,
        out_type=jax.ShapeDtypeStruct((E_PAD * 4,), jnp.float32),
        scratch_types=[
            pltpu.VMEM(((N + 1) * 8,), jnp.float32),
            pltpu.VMEM((16,), jnp.float32),
            pltpu.VMEM((CHUNK,), jnp.int32),
            pltpu.VMEM((CHUNK,), jnp.int32),
            pltpu.VMEM((CHUNK,), jnp.int32),
            pltpu.VMEM((CHUNK,), jnp.int32),
            pltpu.VMEM((CHUNK,), jnp.float32),
            pltpu.VMEM((CHUNK,), jnp.float32),
            pltpu.VMEM((CHUNK * 4,), jnp.float32),
            pltpu.VMEM((CHUNK * 4,), jnp.float32),
            pltpu.SemaphoreType.DMA,
            pltpu.SemaphoreType.DMA,
            pltpu.SemaphoreType.DMA,
            pltpu.SemaphoreType.DMA,
        ],
        compiler_params=pltpu.CompilerParams(needs_layout_passes=False),
    )
    return fn(elr_flat, srcp, dstp, rwdp, c16)


def _sc_edges(h_hbm, src_hbm, dst_hbm, w_hbm,
              accf_hbm, accw_hbm,
              src0, src1, dst0, dst1, sx0, sx1, drow0, drow1,
              gbuf0, gbuf1, stw0, stw1, wb0, wb1,
              accf_sh, accw_sh,
              semi0, semi1, semg0, semg1, semf0, semf1, semw0, semw1):
    sid = lax.axis_index("s")
    lanes = lax.iota(jnp.int32, 16)
    srcb, dstb, gbuf, wb = [src0, src1], [dst0, dst1], [gbuf0, gbuf1], [wb0, wb1]
    sidx, drow, stw = [sx0, sx1], [drow0, drow1], [stw0, stw1]
    semi, semg = [semi0, semi1], [semg0, semg1]
    semf, semw = [semf0, semf1], [semw0, semw1]

    def _zrow(r, carry):
        for jj in range(8):
            stw0[r, pl.ds(jj * 16, 16)] = jnp.zeros((16,), jnp.float32)
            stw1[r, pl.ds(jj * 16, 16)] = jnp.zeros((16,), jnp.float32)
        return carry
    lax.fori_loop(0, CHUNK, _zrow, 0)

    zbase = sid * 640
    for k in range(8):
        pltpu.sync_copy(stw0, accf_sh.at[pl.ds(zbase + k * 80, 80)])

    @pl.when(sid == 0)
    def _():
        for k in range(4):
            pltpu.sync_copy(stw0, accw_sh.at[pl.ds(k * 80, 80)])

    plsc.subcore_barrier()

    tile_base = sid * EDGES_PER_TILE

    def _issue_idx(base, p):
        pltpu.async_copy(src_hbm.at[pl.ds(base, CHUNK)], srcb[p], semi[p])
        pltpu.async_copy(dst_hbm.at[pl.ds(base, CHUNK)], dstb[p], semi[p])
        pltpu.async_copy(w_hbm.at[pl.ds(base * 4, CHUNK * 4)], wb[p], semi[p])

    def _wait_idx(p):
        pltpu.make_async_copy(src_hbm.at[pl.ds(0, CHUNK)], srcb[p], semi[p]).wait()
        pltpu.make_async_copy(dst_hbm.at[pl.ds(0, CHUNK)], dstb[p], semi[p]).wait()
        pltpu.make_async_copy(w_hbm.at[pl.ds(0, CHUNK * 4)], wb[p], semi[p]).wait()

    def _issue_gather(p):
        pltpu.async_copy(h_hbm.at[srcb[p]], gbuf[p], semg[p])

    def _wait_gather(p):
        pltpu.make_async_copy(h_hbm.at[srcb[p]], gbuf[p], semg[p]).wait()

    _issue_idx(tile_base, 0)
    _issue_idx(tile_base + CHUNK, 1)
    _wait_idx(0)
    _issue_gather(0)

    def _wait_accw(p):
        pltpu.make_async_copy(stw[p], accw_sh.at[drow[p]], semw[p]).wait()

    def _pair(i, carry):
        for p in range(2):
            c = 2 * i + p
            dref, g, w = dstb[p], gbuf[p], wb[p]
            _wait_gather(p)

            @pl.when(i > 0)
            def _():
                _wait_accw(p)
                for g4 in range(CHUNK // 4):
                    dvg = sidx[p][pl.ds((g4 // 4) * 16, 16)]
                    idx = ((g4 % 4) * 4) + lax.shift_right_logical(lanes, 2)
                    dv4 = dvg.at[idx].get(mode="promise_in_bounds")
                    e4 = (g4 * 4) + lax.shift_right_logical(lanes, 2)
                    ccol = lax.shift_left(dv4 & 31, 2) + (lanes & 3)
                    plsc.store_scatter(stw[p], [e4, ccol],
                                       jnp.zeros((16,), jnp.float32))

            for gg in range(CHUNK // 16):
                dv = dref[pl.ds(gg * 16, 16)]
                drow[p][pl.ds(gg * 16, 16)] = lax.shift_right_logical(dv, 5)
                sidx[p][pl.ds(gg * 16, 16)] = dv
            for g4 in range(CHUNK // 4):
                dvg = dref[pl.ds((g4 // 4) * 16, 16)]
                idx = ((g4 % 4) * 4) + lax.shift_right_logical(lanes, 2)
                dv4 = dvg.at[idx].get(mode="promise_in_bounds")
                e4 = (g4 * 4) + lax.shift_right_logical(lanes, 2)
                ccol = lax.shift_left(dv4 & 31, 2) + (lanes & 3)
                wv = w[pl.ds(g4 * 16, 16)]
                plsc.store_scatter(stw[p], [e4, ccol], wv)

            for q in range(CHUNK // 4):
                wv = w[pl.ds(q * 16, 16)]
                for qq in range(4):
                    e = q * 4 + qq
                    for j in range(8):
                        g[e, pl.ds(j * 16, 16)] = (
                            g[e, pl.ds(j * 16, 16)] * wv[qq * 4 + j // 2])

            pltpu.async_copy(g, accf_sh.at[sidx[p]], semf[p], add=True)
            pltpu.async_copy(stw[p], accw_sh.at[drow[p]], semw[p], add=True)

            nbase = tile_base + jnp.minimum(c + 2, CHUNKS_PER_TILE - 1) * CHUNK
            _issue_idx(nbase, p)
            if p == 0:
                @pl.when(i > 0)
                def _():
                    pltpu.make_async_copy(
                        gbuf[1], accf_sh.at[sidx[1]], semf[1]).wait()
            else:
                pltpu.make_async_copy(
                    gbuf[0], accf_sh.at[sidx[0]], semf[0]).wait()
            _wait_idx(1 - p)
            _issue_gather(1 - p)
        return carry

    lax.fori_loop(0, CHUNKS_PER_TILE // 2, _pair, 0)
    _wait_gather(0)
    _wait_idx(1)
    pltpu.make_async_copy(gbuf[1], accf_sh.at[sidx[1]], semf[1]).wait()
    _wait_accw(0)
    _wait_accw(1)
    plsc.subcore_barrier()

    for k in range(8):
        pltpu.sync_copy(accf_sh.at[pl.ds(zbase + k * 80, 80)],
                        accf_hbm.at[pl.ds(zbase + k * 80, 80)])

    @pl.when(sid < 8)
    def _():
        ws = sid * 40
        pltpu.sync_copy(accw_sh.at[pl.ds(ws, 40)],
                        accw_hbm.at[pl.ds(ws, 40)])


def _run_sc_edges(h, srcp, dstp, wflat):
    mesh = plsc.VectorSubcoreMesh(
        core_axis_name="c", subcore_axis_name="s", num_cores=1)
    fn = pl.kernel(
        _sc_edges, mesh=mesh,
        out_type=[
            jax.ShapeDtypeStruct((NROWS, HO), jnp.float32),
            jax.ShapeDtypeStruct((WROWS, HO), jnp.float32),
        ],
        scratch_types=[
            pltpu.VMEM((CHUNK,), jnp.int32),
            pltpu.VMEM((CHUNK,), jnp.int32),
            pltpu.VMEM((CHUNK,), jnp.int32),
            pltpu.VMEM((CHUNK,), jnp.int32),
            pltpu.VMEM((CHUNK,), jnp.int32),
            pltpu.VMEM((CHUNK,), jnp.int32),
            pltpu.VMEM((CHUNK,), jnp.int32),
            pltpu.VMEM((CHUNK,), jnp.int32),
            pltpu.VMEM((CHUNK, HO), jnp.float32),
            pltpu.VMEM((CHUNK, HO), jnp.float32),
            pltpu.VMEM((CHUNK, HO), jnp.float32),
            pltpu.VMEM((CHUNK, HO), jnp.float32),
            pltpu.VMEM((CHUNK * 4,), jnp.float32),
            pltpu.VMEM((CHUNK * 4,), jnp.float32),
            pltpu.VMEM_SHARED((NROWS, HO), jnp.float32),
            pltpu.VMEM_SHARED((WROWS, HO), jnp.float32),
            pltpu.SemaphoreType.DMA,
            pltpu.SemaphoreType.DMA,
            pltpu.SemaphoreType.DMA,
            pltpu.SemaphoreType.DMA,
            pltpu.SemaphoreType.DMA,
            pltpu.SemaphoreType.DMA,
            pltpu.SemaphoreType.DMA,
            pltpu.SemaphoreType.DMA,
        ],
        compiler_params=pltpu.CompilerParams(needs_layout_passes=False),
    )
    return fn(h, srcp, dstp, wflat)


def _sc_final(accf_hbm, accw_hbm, uid_hbm, bias_hbm,
              outf_hbm, uout_hbm,
              a0, ob, sw0, uidv, u0, uob, bb, sem):
    cid = lax.axis_index("c")
    sid = lax.axis_index("s")
    wid = sid * 2 + cid
    lanes = lax.iota(jnp.int32, 16)

    pltpu.sync_copy(bias_hbm, bb)
    pltpu.sync_copy(accw_hbm, sw0)
    bm = []
    for j in range(2):
        acc = bb[pl.ds(j * 16, 16)]
        for hh in range(1, H):
            acc = acc + bb[pl.ds(hh * 32 + j * 16, 16)]
        bm.append(acc * 0.25)

    def _do_rows(nrows, aref0, oref, svec_fn):
        for g in range(nrows // 4):
            sv = svec_fn(g)
            ivv = 0.25 / (sv + 1e-9)
            for q in range(4):
                r = g * 4 + q
                for j in range(2):
                    v = bm[j]
                    for hh in range(H):
                        v = v + aref0[r, pl.ds(hh * 32 + j * 16, 16)] * ivv[q * 4 + hh]
                    oref[r, pl.ds(j * 16, 16)] = jnp.maximum(v, 0.0)

    def _svec(n_vec):
        ridx = lax.shift_right_logical(n_vec, 5)
        ccol = lax.shift_left(n_vec & 31, 2) + (lanes & 3)
        return plsc.load_gather(sw0, [ridx, ccol])

    def _svec_nodes(start):
        def f(g):
            n_vec = start + g * 4 + lax.shift_right_logical(lanes, 2)
            return _svec(n_vec)
        return f

    def _nchunk(k, carry):
        start = wid * 320 + k * 64
        pltpu.sync_copy(accf_hbm.at[pl.ds(start, 64)], a0)
        _do_rows(64, a0, ob, _svec_nodes(start))
        pltpu.sync_copy(ob, outf_hbm.at[pl.ds(start, 64)])
        return carry
    lax.fori_loop(0, 5, _nchunk, 0)

    ubase = wid * 32
    pltpu.sync_copy(uid_hbm.at[pl.ds(ubase, 32)], uidv)
    for g in range(2):
        uidv[pl.ds(g * 16, 16)] = uidv[pl.ds(g * 16, 16)] + NUM_ITEMS
    pltpu.async_copy(accf_hbm.at[uidv], u0, sem).wait()

    def _svec_users(g):
        half = uidv[pl.ds((g // 4) * 16, 16)]
        idx = ((g % 4) * 4) + lax.shift_right_logical(lanes, 2)
        n_vec = half.at[idx].get(mode="promise_in_bounds")
        return _svec(n_vec)

    _do_rows(32, u0, uob, _svec_users)
    pltpu.sync_copy(uob, uout_hbm.at[pl.ds(ubase, 32)])


def _run_sc_final(accf, accw, uid_pad, bias):
    mesh = plsc.VectorSubcoreMesh(core_axis_name="c", subcore_axis_name="s")
    fn = pl.kernel(
        _sc_final, mesh=mesh,
        out_type=[
            jax.ShapeDtypeStruct((NROWS, OUT), jnp.float32),
            jax.ShapeDtypeStruct((U_PAD, OUT), jnp.float32),
        ],
        scratch_types=[
            pltpu.VMEM((64, HO), jnp.float32),
            pltpu.VMEM((64, OUT), jnp.float32),
            pltpu.VMEM((WROWS, HO), jnp.float32),
            pltpu.VMEM((32,), jnp.int32),
            pltpu.VMEM((32, HO), jnp.float32),
            pltpu.VMEM((32, OUT), jnp.float32),
            pltpu.VMEM((IN_NODE,), jnp.float32),
            pltpu.SemaphoreType.DMA,
        ],
        compiler_params=pltpu.CompilerParams(needs_layout_passes=False),
    )
    return fn(accf, accw, uid_pad, bias)


def kernel(users_ids, users_features, items_features, edge_index, edge_reward,
           W_node, W_edge, attn_l, attn_r, attn_e, bias):
    x = jnp.concatenate([items_features, users_features], axis=0)
    fl = attn_l.reshape(1, HO)
    fr = attn_r.reshape(1, HO)
    fe = attn_e.reshape(1, HO)

    h, elr, crow = _tc_project(x, W_node, fl, fr, fe, W_edge)

    elr_flat = jnp.pad(elr, ((0, 1), (0, 0))).reshape(-1)
    c16 = crow[0, :16]

    src = edge_index[0]
    dst = edge_index[1]
    pad = E_PAD - E
    srcp = jnp.concatenate([src, jnp.zeros((pad,), src.dtype)])
    dstp = jnp.concatenate([dst, jnp.full((pad,), N, dst.dtype)])
    rwdp = jnp.concatenate([edge_reward, jnp.zeros((pad,), edge_reward.dtype)])

    wflat = _run_sc_weights(elr_flat, srcp, dstp, rwdp, c16)
    accf, accw = _run_sc_edges(h, srcp, dstp, wflat)

    uid_pad = jnp.concatenate(
        [users_ids, jnp.zeros((U_PAD - NUM_USERS,), users_ids.dtype)])
    outf, uout = _run_sc_final(accf, accw, uid_pad, bias)

    return (uout[:NUM_USERS], outf[:NUM_ITEMS])


# speedup vs baseline: 1.0552x; 1.0552x over previous
import jax
import jax.numpy as jnp
from jax import lax
from jax.experimental import pallas as pl
from jax.experimental.pallas import tpu as pltpu
from jax.experimental.pallas import tpu_sc as plsc

NUM_ITEMS = 9000
NUM_USERS = 1000
N = NUM_ITEMS + NUM_USERS
E = 160000
IN_NODE = 128
H = 4
OUT = 32
HO = H * OUT

NT = 16
CHUNK = 80
CHUNKS_PER_TILE = 128
EDGES_PER_TILE = CHUNK * CHUNKS_PER_TILE
E_PAD = EDGES_PER_TILE * NT
NROWS = 10240
WROWS = NROWS // 32
U_PAD = 1024


def _tc_body(x_ref, wn_ref, fl_ref, fr_ref, fe_ref, we_ref,
             h_ref, elr_ref, c_ref):
    xb = x_ref[...]
    wn = wn_ref[...]
    hb = jnp.dot(xb, wn, preferred_element_type=jnp.float32)
    h_ref[...] = hb

    r16 = lax.broadcasted_iota(jnp.int32, (IN_NODE, 8), 0)
    c16 = lax.broadcasted_iota(jnp.int32, (IN_NODE, 8), 1)
    ma = jnp.where((r16 // OUT == c16) & (c16 < H), 1.0, 0.0).astype(jnp.float32)
    mb = jnp.where((r16 // OUT == c16 - H) & (c16 >= H), 1.0, 0.0).astype(jnp.float32)
    tl = hb * fl_ref[...]
    tr = hb * fr_ref[...]
    elr_ref[...] = (jnp.dot(tl, ma, preferred_element_type=jnp.float32)
                    + jnp.dot(tr, mb, preferred_element_type=jnp.float32))

    @pl.when(pl.program_id(0) == 0)
    def _():
        rc = lax.broadcasted_iota(jnp.int32, (IN_NODE, IN_NODE), 0)
        cc = lax.broadcasted_iota(jnp.int32, (IN_NODE, IN_NODE), 1)
        mc = jnp.where((rc // OUT == cc) & (cc < H), 1.0, 0.0).astype(jnp.float32)
        colsum = jnp.sum(we_ref[...], axis=0, keepdims=True)
        ce = colsum * fe_ref[...]
        c_ref[...] = jnp.dot(ce, mc, preferred_element_type=jnp.float32)


def _tc_project(x, wn, fl, fr, fe, we):
    br = 1000
    grid = N // br
    return pl.pallas_call(
        _tc_body,
        grid=(grid,),
        in_specs=[
            pl.BlockSpec((br, IN_NODE), lambda i: (i, 0)),
            pl.BlockSpec((IN_NODE, HO), lambda i: (0, 0)),
            pl.BlockSpec((1, HO), lambda i: (0, 0)),
            pl.BlockSpec((1, HO), lambda i: (0, 0)),
            pl.BlockSpec((1, HO), lambda i: (0, 0)),
            pl.BlockSpec((16, HO), lambda i: (0, 0)),
        ],
        out_specs=[
            pl.BlockSpec((br, IN_NODE), lambda i: (i, 0)),
            pl.BlockSpec((br, 8), lambda i: (i, 0)),
            pl.BlockSpec((1, IN_NODE), lambda i: (0, 0)),
        ],
        out_shape=[
            jax.ShapeDtypeStruct((N, IN_NODE), jnp.float32),
            jax.ShapeDtypeStruct((N, 8), jnp.float32),
            jax.ShapeDtypeStruct((1, IN_NODE), jnp.float32),
        ],
    )(x, wn, fl, fr, fe, we)


def _sc_weights(elr_hbm, src_hbm, dst_hbm, rwd_hbm, c_hbm, w_hbm,
                elr_v, c_v, src0, src1, dst0, dst1, rwd0, rwd1, wc0, wc1,
                semi0, semi1, semo0, semo1):
    wid = lax.axis_index("s") * 2 + lax.axis_index("c")
    lanes = lax.iota(jnp.int32, 16)
    srcb, dstb, rwdb, wc = [src0, src1], [dst0, dst1], [rwd0, rwd1], [wc0, wc1]
    semi, semo = [semi0, semi1], [semo0, semo1]

    pltpu.sync_copy(elr_hbm, elr_v)
    pltpu.sync_copy(c_hbm, c_v)
    cvec = c_v[pl.ds(0, 16)]
    c_sc = [cvec[hh] for hh in range(H)]

    ept = EDGES_PER_TILE // 2
    cpt = CHUNKS_PER_TILE // 2
    tile_base = wid * ept

    def _issue_idx(base, p):
        pltpu.async_copy(src_hbm.at[pl.ds(base, CHUNK)], srcb[p], semi[p])
        pltpu.async_copy(dst_hbm.at[pl.ds(base, CHUNK)], dstb[p], semi[p])
        pltpu.async_copy(rwd_hbm.at[pl.ds(base, CHUNK)], rwdb[p], semi[p])

    def _wait_idx(p):
        pltpu.make_async_copy(src_hbm.at[pl.ds(0, CHUNK)], srcb[p], semi[p]).wait()
        pltpu.make_async_copy(dst_hbm.at[pl.ds(0, CHUNK)], dstb[p], semi[p]).wait()
        pltpu.make_async_copy(rwd_hbm.at[pl.ds(0, CHUNK)], rwdb[p], semi[p]).wait()

    def _drain_wout(p):
        pltpu.make_async_copy(wc[p], w_hbm.at[pl.ds(0, CHUNK * 4)], semo[p]).wait()

    _issue_idx(tile_base, 0)
    _issue_idx(tile_base + CHUNK, 1)

    def _pair(i, carry):
        for p in range(2):
            c = 2 * i + p
            _wait_idx(p)

            @pl.when(i > 0)
            def _():
                _drain_wout(p)

            for g in range(CHUNK // 16):
                sv = srcb[p][pl.ds(g * 16, 16)]
                dv = dstb[p][pl.ds(g * 16, 16)]
                rv = rwdb[p][pl.ds(g * 16, 16)]
                e_vec = lanes + (g * 16)
                for hh in range(H):
                    eli = plsc.load_gather(elr_v, [sv * 8 + hh])
                    eri = plsc.load_gather(elr_v, [dv * 8 + (4 + hh)])
                    z = eli + eri + rv * c_sc[hh]
                    z = jnp.where(z > 0.0, z, z * 0.2)
                    w = jnp.exp(z)
                    plsc.store_scatter(wc[p], [e_vec * 4 + hh], w)

            base = tile_base + c * CHUNK
            pltpu.async_copy(wc[p], w_hbm.at[pl.ds(base * 4, CHUNK * 4)], semo[p])
            nbase = tile_base + jnp.minimum(c + 2, cpt - 1) * CHUNK
            _issue_idx(nbase, p)
        return carry

    lax.fori_loop(0, cpt // 2, _pair, 0)
    _drain_wout(0)
    _drain_wout(1)
    _wait_idx(0)
    _wait_idx(1)


def _run_sc_weights(elr_flat, srcp, dstp, rwdp, c16):
    mesh = plsc.VectorSubcoreMesh(core_axis_name="c", subcore_axis_name="s")
    fn = pl.kernel(
        _sc_weights, mesh=mesh,
        out_type=jax.ShapeDtypeStruct((E_PAD * 4,), jnp.float32),
        scratch_types=[
            pltpu.VMEM(((N + 1) * 8,), jnp.float32),
            pltpu.VMEM((16,), jnp.float32),
            pltpu.VMEM((CHUNK,), jnp.int32),
            pltpu.VMEM((CHUNK,), jnp.int32),
            pltpu.VMEM((CHUNK,), jnp.int32),
            pltpu.VMEM((CHUNK,), jnp.int32),
            pltpu.VMEM((CHUNK,), jnp.float32),
            pltpu.VMEM((CHUNK,), jnp.float32),
            pltpu.VMEM((CHUNK * 4,), jnp.float32),
            pltpu.VMEM((CHUNK * 4,), jnp.float32),
            pltpu.SemaphoreType.DMA,
            pltpu.SemaphoreType.DMA,
            pltpu.SemaphoreType.DMA,
            pltpu.SemaphoreType.DMA,
        ],
        compiler_params=pltpu.CompilerParams(needs_layout_passes=False),
    )
    return fn(elr_flat, srcp, dstp, rwdp, c16)


def _sc_edges(h_hbm, src_hbm, dst_hbm, w_hbm,
              accf_hbm, accw_hbm,
              src0, src1, dst0, dst1, sx0, sx1, drow0, drow1,
              gbuf0, gbuf1, stw0, stw1, wb0, wb1,
              accf_sh, accw_sh,
              semi0, semi1, semg0, semg1, semf0, semf1, semw0, semw1):
    sid = lax.axis_index("s")
    lanes = lax.iota(jnp.int32, 16)
    srcb, dstb, gbuf, wb = [src0, src1], [dst0, dst1], [gbuf0, gbuf1], [wb0, wb1]
    sidx, drow, stw = [sx0, sx1], [drow0, drow1], [stw0, stw1]
    semi, semg = [semi0, semi1], [semg0, semg1]
    semf, semw = [semf0, semf1], [semw0, semw1]

    def _zrow(r, carry):
        for jj in range(8):
            stw0[r, pl.ds(jj * 16, 16)] = jnp.zeros((16,), jnp.float32)
            stw1[r, pl.ds(jj * 16, 16)] = jnp.zeros((16,), jnp.float32)
        return carry
    lax.fori_loop(0, CHUNK, _zrow, 0)

    zbase = sid * 640
    for k in range(8):
        pltpu.sync_copy(stw0, accf_sh.at[pl.ds(zbase + k * 80, 80)])

    @pl.when(sid == 0)
    def _():
        for k in range(4):
            pltpu.sync_copy(stw0, accw_sh.at[pl.ds(k * 80, 80)])

    plsc.subcore_barrier()

    tile_base = sid * EDGES_PER_TILE

    def _issue_idx(base, p):
        pltpu.async_copy(src_hbm.at[pl.ds(base, CHUNK)], srcb[p], semi[p])
        pltpu.async_copy(dst_hbm.at[pl.ds(base, CHUNK)], dstb[p], semi[p])
        pltpu.async_copy(w_hbm.at[pl.ds(base * 4, CHUNK * 4)], wb[p], semi[p])

    def _wait_idx(p):
        pltpu.make_async_copy(src_hbm.at[pl.ds(0, CHUNK)], srcb[p], semi[p]).wait()
        pltpu.make_async_copy(dst_hbm.at[pl.ds(0, CHUNK)], dstb[p], semi[p]).wait()
        pltpu.make_async_copy(w_hbm.at[pl.ds(0, CHUNK * 4)], wb[p], semi[p]).wait()

    def _issue_gather(p):
        pltpu.async_copy(h_hbm.at[srcb[p]], gbuf[p], semg[p])

    def _wait_gather(p):
        pltpu.make_async_copy(h_hbm.at[srcb[p]], gbuf[p], semg[p]).wait()

    _issue_idx(tile_base, 0)
    _issue_idx(tile_base + CHUNK, 1)
    _wait_idx(0)
    _issue_gather(0)

    def _wait_accw(p):
        pltpu.make_async_copy(stw[p], accw_sh.at[drow[p]], semw[p]).wait()

    def _pair(i, carry):
        for p in range(2):
            c = 2 * i + p
            dref, g, w = dstb[p], gbuf[p], wb[p]
            _wait_gather(p)

            @pl.when(i > 0)
            def _():
                _wait_accw(p)
                for g4 in range(CHUNK // 4):
                    dvg = sidx[p][pl.ds((g4 // 4) * 16, 16)]
                    idx = ((g4 % 4) * 4) + lax.shift_right_logical(lanes, 2)
                    dv4 = dvg.at[idx].get(mode="promise_in_bounds")
                    e4 = (g4 * 4) + lax.shift_right_logical(lanes, 2)
                    ccol = lax.shift_left(dv4 & 31, 2) + (lanes & 3)
                    plsc.store_scatter(stw[p], [e4, ccol],
                                       jnp.zeros((16,), jnp.float32))

            for gg in range(CHUNK // 16):
                dv = dref[pl.ds(gg * 16, 16)]
                drow[p][pl.ds(gg * 16, 16)] = lax.shift_right_logical(dv, 5)
                sidx[p][pl.ds(gg * 16, 16)] = dv
            for g4 in range(CHUNK // 4):
                dvg = dref[pl.ds((g4 // 4) * 16, 16)]
                idx = ((g4 % 4) * 4) + lax.shift_right_logical(lanes, 2)
                dv4 = dvg.at[idx].get(mode="promise_in_bounds")
                e4 = (g4 * 4) + lax.shift_right_logical(lanes, 2)
                ccol = lax.shift_left(dv4 & 31, 2) + (lanes & 3)
                wv = w[pl.ds(g4 * 16, 16)]
                plsc.store_scatter(stw[p], [e4, ccol], wv)

            def _edges4(q, carry2, g=g, w=w):
                wv = w[pl.ds(q * 16, 16)]
                for qq in range(4):
                    e = q * 4 + qq
                    for j in range(8):
                        g[e, pl.ds(j * 16, 16)] = (
                            g[e, pl.ds(j * 16, 16)] * wv[qq * 4 + j // 2])
                return carry2
            lax.fori_loop(0, CHUNK // 4, _edges4, 0)

            pltpu.async_copy(g, accf_sh.at[sidx[p]], semf[p], add=True)
            pltpu.async_copy(stw[p], accw_sh.at[drow[p]], semw[p], add=True)

            nbase = tile_base + jnp.minimum(c + 2, CHUNKS_PER_TILE - 1) * CHUNK
            _issue_idx(nbase, p)
            if p == 0:
                @pl.when(i > 0)
                def _():
                    pltpu.make_async_copy(
                        gbuf[1], accf_sh.at[sidx[1]], semf[1]).wait()
            else:
                pltpu.make_async_copy(
                    gbuf[0], accf_sh.at[sidx[0]], semf[0]).wait()
            _wait_idx(1 - p)
            _issue_gather(1 - p)
        return carry

    lax.fori_loop(0, CHUNKS_PER_TILE // 2, _pair, 0)
    _wait_gather(0)
    _wait_idx(1)
    pltpu.make_async_copy(gbuf[1], accf_sh.at[sidx[1]], semf[1]).wait()
    _wait_accw(0)
    _wait_accw(1)
    plsc.subcore_barrier()

    for k in range(8):
        pltpu.sync_copy(accf_sh.at[pl.ds(zbase + k * 80, 80)],
                        accf_hbm.at[pl.ds(zbase + k * 80, 80)])

    @pl.when(sid < 8)
    def _():
        ws = sid * 40
        pltpu.sync_copy(accw_sh.at[pl.ds(ws, 40)],
                        accw_hbm.at[pl.ds(ws, 40)])


def _run_sc_edges(h, srcp, dstp, wflat):
    mesh = plsc.VectorSubcoreMesh(
        core_axis_name="c", subcore_axis_name="s", num_cores=1)
    fn = pl.kernel(
        _sc_edges, mesh=mesh,
        out_type=[
            jax.ShapeDtypeStruct((NROWS, HO), jnp.float32),
            jax.ShapeDtypeStruct((WROWS, HO), jnp.float32),
        ],
        scratch_types=[
            pltpu.VMEM((CHUNK,), jnp.int32),
            pltpu.VMEM((CHUNK,), jnp.int32),
            pltpu.VMEM((CHUNK,), jnp.int32),
            pltpu.VMEM((CHUNK,), jnp.int32),
            pltpu.VMEM((CHUNK,), jnp.int32),
            pltpu.VMEM((CHUNK,), jnp.int32),
            pltpu.VMEM((CHUNK,), jnp.int32),
            pltpu.VMEM((CHUNK,), jnp.int32),
            pltpu.VMEM((CHUNK, HO), jnp.float32),
            pltpu.VMEM((CHUNK, HO), jnp.float32),
            pltpu.VMEM((CHUNK, HO), jnp.float32),
            pltpu.VMEM((CHUNK, HO), jnp.float32),
            pltpu.VMEM((CHUNK * 4,), jnp.float32),
            pltpu.VMEM((CHUNK * 4,), jnp.float32),
            pltpu.VMEM_SHARED((NROWS, HO), jnp.float32),
            pltpu.VMEM_SHARED((WROWS, HO), jnp.float32),
            pltpu.SemaphoreType.DMA,
            pltpu.SemaphoreType.DMA,
            pltpu.SemaphoreType.DMA,
            pltpu.SemaphoreType.DMA,
            pltpu.SemaphoreType.DMA,
            pltpu.SemaphoreType.DMA,
            pltpu.SemaphoreType.DMA,
            pltpu.SemaphoreType.DMA,
        ],
        compiler_params=pltpu.CompilerParams(needs_layout_passes=False),
    )
    return fn(h, srcp, dstp, wflat)


def _sc_final(accf_hbm, accw_hbm, uid_hbm, bias_hbm,
              outf_hbm, uout_hbm,
              a0, ob, sw0, uidv, u0, uob, bb, sem):
    cid = lax.axis_index("c")
    sid = lax.axis_index("s")
    wid = sid * 2 + cid
    lanes = lax.iota(jnp.int32, 16)

    pltpu.sync_copy(bias_hbm, bb)
    pltpu.sync_copy(accw_hbm, sw0)
    bm = []
    for j in range(2):
        acc = bb[pl.ds(j * 16, 16)]
        for hh in range(1, H):
            acc = acc + bb[pl.ds(hh * 32 + j * 16, 16)]
        bm.append(acc * 0.25)

    def _do_rows(nrows, aref0, oref, svec_fn):
        for g in range(nrows // 4):
            sv = svec_fn(g)
            ivv = 0.25 / (sv + 1e-9)
            for q in range(4):
                r = g * 4 + q
                for j in range(2):
                    v = bm[j]
                    for hh in range(H):
                        v = v + aref0[r, pl.ds(hh * 32 + j * 16, 16)] * ivv[q * 4 + hh]
                    oref[r, pl.ds(j * 16, 16)] = jnp.maximum(v, 0.0)

    def _svec(n_vec):
        ridx = lax.shift_right_logical(n_vec, 5)
        ccol = lax.shift_left(n_vec & 31, 2) + (lanes & 3)
        return plsc.load_gather(sw0, [ridx, ccol])

    def _svec_nodes(start):
        def f(g):
            n_vec = start + g * 4 + lax.shift_right_logical(lanes, 2)
            return _svec(n_vec)
        return f

    def _nchunk(k, carry):
        start = wid * 320 + k * 64
        pltpu.sync_copy(accf_hbm.at[pl.ds(start, 64)], a0)
        _do_rows(64, a0, ob, _svec_nodes(start))
        pltpu.sync_copy(ob, outf_hbm.at[pl.ds(start, 64)])
        return carry
    lax.fori_loop(0, 5, _nchunk, 0)

    ubase = wid * 32
    pltpu.sync_copy(uid_hbm.at[pl.ds(ubase, 32)], uidv)
    for g in range(2):
        uidv[pl.ds(g * 16, 16)] = uidv[pl.ds(g * 16, 16)] + NUM_ITEMS
    pltpu.async_copy(accf_hbm.at[uidv], u0, sem).wait()

    def _svec_users(g):
        half = uidv[pl.ds((g // 4) * 16, 16)]
        idx = ((g % 4) * 4) + lax.shift_right_logical(lanes, 2)
        n_vec = half.at[idx].get(mode="promise_in_bounds")
        return _svec(n_vec)

    _do_rows(32, u0, uob, _svec_users)
    pltpu.sync_copy(uob, uout_hbm.at[pl.ds(ubase, 32)])


def _run_sc_final(accf, accw, uid_pad, bias):
    mesh = plsc.VectorSubcoreMesh(core_axis_name="c", subcore_axis_name="s")
    fn = pl.kernel(
        _sc_final, mesh=mesh,
        out_type=[
            jax.ShapeDtypeStruct((NROWS, OUT), jnp.float32),
            jax.ShapeDtypeStruct((U_PAD, OUT), jnp.float32),
        ],
        scratch_types=[
            pltpu.VMEM((64, HO), jnp.float32),
            pltpu.VMEM((64, OUT), jnp.float32),
            pltpu.VMEM((WROWS, HO), jnp.float32),
            pltpu.VMEM((32,), jnp.int32),
            pltpu.VMEM((32, HO), jnp.float32),
            pltpu.VMEM((32, OUT), jnp.float32),
            pltpu.VMEM((IN_NODE,), jnp.float32),
            pltpu.SemaphoreType.DMA,
        ],
        compiler_params=pltpu.CompilerParams(needs_layout_passes=False),
    )
    return fn(accf, accw, uid_pad, bias)


def kernel(users_ids, users_features, items_features, edge_index, edge_reward,
           W_node, W_edge, attn_l, attn_r, attn_e, bias):
    x = jnp.concatenate([items_features, users_features], axis=0)
    fl = attn_l.reshape(1, HO)
    fr = attn_r.reshape(1, HO)
    fe = attn_e.reshape(1, HO)

    h, elr, crow = _tc_project(x, W_node, fl, fr, fe, W_edge)

    elr_flat = jnp.pad(elr, ((0, 1), (0, 0))).reshape(-1)
    c16 = crow[0, :16]

    src = edge_index[0]
    dst = edge_index[1]
    pad = E_PAD - E
    srcp = jnp.concatenate([src, jnp.zeros((pad,), src.dtype)])
    dstp = jnp.concatenate([dst, jnp.full((pad,), N, dst.dtype)])
    rwdp = jnp.concatenate([edge_reward, jnp.zeros((pad,), edge_reward.dtype)])

    wflat = _run_sc_weights(elr_flat, srcp, dstp, rwdp, c16)
    accf, accw = _run_sc_edges(h, srcp, dstp, wflat)

    uid_pad = jnp.concatenate(
        [users_ids, jnp.zeros((U_PAD - NUM_USERS,), users_ids.dtype)])
    outf, uout = _run_sc_final(accf, accw, uid_pad, bias)

    return (uout[:NUM_USERS], outf[:NUM_ITEMS])


# speedup vs baseline: 1.0742x; 1.0179x over previous
import jax
import jax.numpy as jnp
from jax import lax
from jax.experimental import pallas as pl
from jax.experimental.pallas import tpu as pltpu
from jax.experimental.pallas import tpu_sc as plsc

NUM_ITEMS = 9000
NUM_USERS = 1000
N = NUM_ITEMS + NUM_USERS
E = 160000
IN_NODE = 128
H = 4
OUT = 32
HO = H * OUT

NT = 16
CHUNK = 80
CHUNKS_PER_TILE = 128
EDGES_PER_TILE = CHUNK * CHUNKS_PER_TILE
E_PAD = EDGES_PER_TILE * NT
NROWS = 10240
WROWS = NROWS // 32
U_PAD = 1024


def _tc_body(x_ref, wn_ref, fl_ref, fr_ref, fe_ref, we_ref,
             h_ref, elr_ref, c_ref):
    xb = x_ref[...]
    wn = wn_ref[...]
    hb = jnp.dot(xb, wn, preferred_element_type=jnp.float32)
    h_ref[...] = hb

    r16 = lax.broadcasted_iota(jnp.int32, (IN_NODE, 8), 0)
    c16 = lax.broadcasted_iota(jnp.int32, (IN_NODE, 8), 1)
    ma = jnp.where((r16 // OUT == c16) & (c16 < H), 1.0, 0.0).astype(jnp.float32)
    mb = jnp.where((r16 // OUT == c16 - H) & (c16 >= H), 1.0, 0.0).astype(jnp.float32)
    tl = hb * fl_ref[...]
    tr = hb * fr_ref[...]
    elr_ref[...] = (jnp.dot(tl, ma, preferred_element_type=jnp.float32)
                    + jnp.dot(tr, mb, preferred_element_type=jnp.float32))

    @pl.when(pl.program_id(0) == 0)
    def _():
        rc = lax.broadcasted_iota(jnp.int32, (IN_NODE, IN_NODE), 0)
        cc = lax.broadcasted_iota(jnp.int32, (IN_NODE, IN_NODE), 1)
        mc = jnp.where((rc // OUT == cc) & (cc < H), 1.0, 0.0).astype(jnp.float32)
        colsum = jnp.sum(we_ref[...], axis=0, keepdims=True)
        ce = colsum * fe_ref[...]
        c_ref[...] = jnp.dot(ce, mc, preferred_element_type=jnp.float32)


def _tc_project(x, wn, fl, fr, fe, we):
    br = 1000
    grid = N // br
    return pl.pallas_call(
        _tc_body,
        grid=(grid,),
        in_specs=[
            pl.BlockSpec((br, IN_NODE), lambda i: (i, 0)),
            pl.BlockSpec((IN_NODE, HO), lambda i: (0, 0)),
            pl.BlockSpec((1, HO), lambda i: (0, 0)),
            pl.BlockSpec((1, HO), lambda i: (0, 0)),
            pl.BlockSpec((1, HO), lambda i: (0, 0)),
            pl.BlockSpec((16, HO), lambda i: (0, 0)),
        ],
        out_specs=[
            pl.BlockSpec((br, IN_NODE), lambda i: (i, 0)),
            pl.BlockSpec((br, 8), lambda i: (i, 0)),
            pl.BlockSpec((1, IN_NODE), lambda i: (0, 0)),
        ],
        out_shape=[
            jax.ShapeDtypeStruct((N, IN_NODE), jnp.float32),
            jax.ShapeDtypeStruct((N, 8), jnp.float32),
            jax.ShapeDtypeStruct((1, IN_NODE), jnp.float32),
        ],
    )(x, wn, fl, fr, fe, we)


def _sc_weights(elr_hbm, src_hbm, dst_hbm, rwd_hbm, c_hbm,
                w_hbm, accw0_hbm, accw1_hbm,
                elr_v, c_v, src0, src1, dst0, dst1, rwd0, rwd1, wc0, wc1,
                stw, drow_v, accw_sh,
                semi0, semi1, semo0, semo1):
    cid = lax.axis_index("c")
    sid = lax.axis_index("s")
    wid = sid * 2 + cid
    lanes = lax.iota(jnp.int32, 16)
    srcb, dstb, rwdb, wc = [src0, src1], [dst0, dst1], [rwd0, rwd1], [wc0, wc1]
    semi, semo = [semi0, semi1], [semo0, semo1]

    pltpu.sync_copy(elr_hbm, elr_v)
    pltpu.sync_copy(c_hbm, c_v)
    cvec = c_v[pl.ds(0, 16)]
    c_sc = [cvec[hh] for hh in range(H)]

    def _zrow(r, carry):
        for jj in range(8):
            stw[r, pl.ds(jj * 16, 16)] = jnp.zeros((16,), jnp.float32)
        return carry
    lax.fori_loop(0, CHUNK, _zrow, 0)

    @pl.when(sid == 0)
    def _():
        for k in range(4):
            pltpu.sync_copy(stw, accw_sh.at[pl.ds(k * 80, 80)])

    plsc.subcore_barrier()

    ept = EDGES_PER_TILE // 2
    cpt = CHUNKS_PER_TILE // 2
    tile_base = wid * ept

    def _issue_idx(base, p):
        pltpu.async_copy(src_hbm.at[pl.ds(base, CHUNK)], srcb[p], semi[p])
        pltpu.async_copy(dst_hbm.at[pl.ds(base, CHUNK)], dstb[p], semi[p])
        pltpu.async_copy(rwd_hbm.at[pl.ds(base, CHUNK)], rwdb[p], semi[p])

    def _wait_idx(p):
        pltpu.make_async_copy(src_hbm.at[pl.ds(0, CHUNK)], srcb[p], semi[p]).wait()
        pltpu.make_async_copy(dst_hbm.at[pl.ds(0, CHUNK)], dstb[p], semi[p]).wait()
        pltpu.make_async_copy(rwd_hbm.at[pl.ds(0, CHUNK)], rwdb[p], semi[p]).wait()

    def _drain_wout(p):
        pltpu.make_async_copy(wc[p], w_hbm.at[pl.ds(0, CHUNK * 4)], semo[p]).wait()

    _issue_idx(tile_base, 0)
    _issue_idx(tile_base + CHUNK, 1)

    def _pair(i, carry):
        for p in range(2):
            c = 2 * i + p
            _wait_idx(p)

            @pl.when(i > 0)
            def _():
                _drain_wout(p)

            for g in range(CHUNK // 16):
                sv = srcb[p][pl.ds(g * 16, 16)]
                dv = dstb[p][pl.ds(g * 16, 16)]
                rv = rwdb[p][pl.ds(g * 16, 16)]
                e_vec = lanes + (g * 16)
                drow_v[pl.ds(g * 16, 16)] = lax.shift_right_logical(dv, 5)
                dcol = lax.shift_left(dv & 31, 2)
                for hh in range(H):
                    eli = plsc.load_gather(elr_v, [sv * 8 + hh])
                    eri = plsc.load_gather(elr_v, [dv * 8 + (4 + hh)])
                    z = eli + eri + rv * c_sc[hh]
                    z = jnp.where(z > 0.0, z, z * 0.2)
                    w = jnp.exp(z)
                    plsc.store_scatter(wc[p], [e_vec * 4 + hh], w)
                    plsc.store_scatter(stw, [e_vec, dcol + hh], w)

            pltpu.sync_copy(stw, accw_sh.at[drow_v], add=True)

            for g in range(CHUNK // 16):
                dv = dstb[p][pl.ds(g * 16, 16)]
                e_vec = lanes + (g * 16)
                dcol = lax.shift_left(dv & 31, 2)
                for hh in range(H):
                    plsc.store_scatter(stw, [e_vec, dcol + hh],
                                       jnp.zeros((16,), jnp.float32))

            base = tile_base + c * CHUNK
            pltpu.async_copy(wc[p], w_hbm.at[pl.ds(base * 4, CHUNK * 4)], semo[p])
            nbase = tile_base + jnp.minimum(c + 2, cpt - 1) * CHUNK
            _issue_idx(nbase, p)
        return carry

    lax.fori_loop(0, cpt // 2, _pair, 0)
    _drain_wout(0)
    _drain_wout(1)
    _wait_idx(0)
    _wait_idx(1)
    plsc.subcore_barrier()

    @pl.when(sid < 8)
    def _():
        ws = sid * 40

        @pl.when(cid == 0)
        def _():
            pltpu.sync_copy(accw_sh.at[pl.ds(ws, 40)],
                            accw0_hbm.at[pl.ds(ws, 40)])

        @pl.when(cid == 1)
        def _():
            pltpu.sync_copy(accw_sh.at[pl.ds(ws, 40)],
                            accw1_hbm.at[pl.ds(ws, 40)])


def _run_sc_weights(elr_flat, srcp, dstp, rwdp, c16):
    mesh = plsc.VectorSubcoreMesh(core_axis_name="c", subcore_axis_name="s")
    fn = pl.kernel(
        _sc_weights, mesh=mesh,
        out_type=[
            jax.ShapeDtypeStruct((E_PAD * 4,), jnp.float32),
            jax.ShapeDtypeStruct((WROWS, HO), jnp.float32),
            jax.ShapeDtypeStruct((WROWS, HO), jnp.float32),
        ],
        scratch_types=[
            pltpu.VMEM(((N + 1) * 8,), jnp.float32),
            pltpu.VMEM((16,), jnp.float32),
            pltpu.VMEM((CHUNK,), jnp.int32),
            pltpu.VMEM((CHUNK,), jnp.int32),
            pltpu.VMEM((CHUNK,), jnp.int32),
            pltpu.VMEM((CHUNK,), jnp.int32),
            pltpu.VMEM((CHUNK,), jnp.float32),
            pltpu.VMEM((CHUNK,), jnp.float32),
            pltpu.VMEM((CHUNK * 4,), jnp.float32),
            pltpu.VMEM((CHUNK * 4,), jnp.float32),
            pltpu.VMEM((CHUNK, HO), jnp.float32),
            pltpu.VMEM((CHUNK,), jnp.int32),
            pltpu.VMEM_SHARED((WROWS, HO), jnp.float32),
            pltpu.SemaphoreType.DMA,
            pltpu.SemaphoreType.DMA,
            pltpu.SemaphoreType.DMA,
            pltpu.SemaphoreType.DMA,
        ],
        compiler_params=pltpu.CompilerParams(needs_layout_passes=False),
    )
    return fn(elr_flat, srcp, dstp, rwdp, c16)


def _sc_edges(h_hbm, src_hbm, dst_hbm, w_hbm,
              accf_hbm,
              src0, src1, dst0, dst1, sx0, sx1,
              gbuf0, gbuf1, wb0, wb1,
              accf_sh,
              semi0, semi1, semg0, semg1, semf0, semf1):
    sid = lax.axis_index("s")
    srcb, dstb, gbuf, wb = [src0, src1], [dst0, dst1], [gbuf0, gbuf1], [wb0, wb1]
    sidx = [sx0, sx1]
    semi, semg = [semi0, semi1], [semg0, semg1]
    semf = [semf0, semf1]

    def _zrow(r, carry):
        for jj in range(8):
            gbuf0[r, pl.ds(jj * 16, 16)] = jnp.zeros((16,), jnp.float32)
        return carry
    lax.fori_loop(0, CHUNK, _zrow, 0)
    zbase = sid * 640
    for k in range(8):
        pltpu.sync_copy(gbuf0, accf_sh.at[pl.ds(zbase + k * 80, 80)])

    plsc.subcore_barrier()

    tile_base = sid * EDGES_PER_TILE

    def _issue_idx(base, p):
        pltpu.async_copy(src_hbm.at[pl.ds(base, CHUNK)], srcb[p], semi[p])
        pltpu.async_copy(dst_hbm.at[pl.ds(base, CHUNK)], dstb[p], semi[p])
        pltpu.async_copy(w_hbm.at[pl.ds(base * 4, CHUNK * 4)], wb[p], semi[p])

    def _wait_idx(p):
        pltpu.make_async_copy(src_hbm.at[pl.ds(0, CHUNK)], srcb[p], semi[p]).wait()
        pltpu.make_async_copy(dst_hbm.at[pl.ds(0, CHUNK)], dstb[p], semi[p]).wait()
        pltpu.make_async_copy(w_hbm.at[pl.ds(0, CHUNK * 4)], wb[p], semi[p]).wait()

    def _issue_gather(p):
        pltpu.async_copy(h_hbm.at[srcb[p]], gbuf[p], semg[p])

    def _wait_gather(p):
        pltpu.make_async_copy(h_hbm.at[srcb[p]], gbuf[p], semg[p]).wait()

    _issue_idx(tile_base, 0)
    _issue_idx(tile_base + CHUNK, 1)
    _wait_idx(0)
    _issue_gather(0)

    def _pair(i, carry):
        for p in range(2):
            c = 2 * i + p
            dref, g, w = dstb[p], gbuf[p], wb[p]
            _wait_gather(p)

            for gg in range(CHUNK // 16):
                sidx[p][pl.ds(gg * 16, 16)] = dref[pl.ds(gg * 16, 16)]

            def _edges4(q, carry2, g=g, w=w):
                wv = w[pl.ds(q * 16, 16)]
                for qq in range(4):
                    e = q * 4 + qq
                    for j in range(8):
                        g[e, pl.ds(j * 16, 16)] = (
                            g[e, pl.ds(j * 16, 16)] * wv[qq * 4 + j // 2])
                return carry2
            lax.fori_loop(0, CHUNK // 4, _edges4, 0)

            pltpu.async_copy(g, accf_sh.at[sidx[p]], semf[p], add=True)

            nbase = tile_base + jnp.minimum(c + 2, CHUNKS_PER_TILE - 1) * CHUNK
            _issue_idx(nbase, p)
            if p == 0:
                @pl.when(i > 0)
                def _():
                    pltpu.make_async_copy(
                        gbuf[1], accf_sh.at[sidx[1]], semf[1]).wait()
            else:
                pltpu.make_async_copy(
                    gbuf[0], accf_sh.at[sidx[0]], semf[0]).wait()
            _wait_idx(1 - p)
            _issue_gather(1 - p)
        return carry

    lax.fori_loop(0, CHUNKS_PER_TILE // 2, _pair, 0)
    _wait_gather(0)
    _wait_idx(1)
    pltpu.make_async_copy(gbuf[1], accf_sh.at[sidx[1]], semf[1]).wait()
    plsc.subcore_barrier()

    for k in range(8):
        pltpu.sync_copy(accf_sh.at[pl.ds(zbase + k * 80, 80)],
                        accf_hbm.at[pl.ds(zbase + k * 80, 80)])


def _run_sc_edges(h, srcp, dstp, wflat):
    mesh = plsc.VectorSubcoreMesh(
        core_axis_name="c", subcore_axis_name="s", num_cores=1)
    fn = pl.kernel(
        _sc_edges, mesh=mesh,
        out_type=jax.ShapeDtypeStruct((NROWS, HO), jnp.float32),
        scratch_types=[
            pltpu.VMEM((CHUNK,), jnp.int32),
            pltpu.VMEM((CHUNK,), jnp.int32),
            pltpu.VMEM((CHUNK,), jnp.int32),
            pltpu.VMEM((CHUNK,), jnp.int32),
            pltpu.VMEM((CHUNK,), jnp.int32),
            pltpu.VMEM((CHUNK,), jnp.int32),
            pltpu.VMEM((CHUNK, HO), jnp.float32),
            pltpu.VMEM((CHUNK, HO), jnp.float32),
            pltpu.VMEM((CHUNK * 4,), jnp.float32),
            pltpu.VMEM((CHUNK * 4,), jnp.float32),
            pltpu.VMEM_SHARED((NROWS, HO), jnp.float32),
            pltpu.SemaphoreType.DMA,
            pltpu.SemaphoreType.DMA,
            pltpu.SemaphoreType.DMA,
            pltpu.SemaphoreType.DMA,
            pltpu.SemaphoreType.DMA,
            pltpu.SemaphoreType.DMA,
        ],
        compiler_params=pltpu.CompilerParams(needs_layout_passes=False),
    )
    return fn(h, srcp, dstp, wflat)


def _sc_final(accf_hbm, accw0_hbm, accw1_hbm, uid_hbm, bias_hbm,
              outf_hbm, uout_hbm,
              a0, ob, sw0, sw1, uidv, u0, uob, bb, sem):
    cid = lax.axis_index("c")
    sid = lax.axis_index("s")
    wid = sid * 2 + cid
    lanes = lax.iota(jnp.int32, 16)

    pltpu.sync_copy(bias_hbm, bb)
    pltpu.sync_copy(accw0_hbm, sw0)
    pltpu.sync_copy(accw1_hbm, sw1)
    bm = []
    for j in range(2):
        acc = bb[pl.ds(j * 16, 16)]
        for hh in range(1, H):
            acc = acc + bb[pl.ds(hh * 32 + j * 16, 16)]
        bm.append(acc * 0.25)

    def _do_rows(nrows, aref0, oref, svec_fn):
        for g in range(nrows // 4):
            sv = svec_fn(g)
            ivv = 0.25 / (sv + 1e-9)
            for q in range(4):
                r = g * 4 + q
                for j in range(2):
                    v = bm[j]
                    for hh in range(H):
                        v = v + aref0[r, pl.ds(hh * 32 + j * 16, 16)] * ivv[q * 4 + hh]
                    oref[r, pl.ds(j * 16, 16)] = jnp.maximum(v, 0.0)

    def _svec(n_vec):
        ridx = lax.shift_right_logical(n_vec, 5)
        ccol = lax.shift_left(n_vec & 31, 2) + (lanes & 3)
        return (plsc.load_gather(sw0, [ridx, ccol])
                + plsc.load_gather(sw1, [ridx, ccol]))

    def _svec_nodes(start):
        def f(g):
            n_vec = start + g * 4 + lax.shift_right_logical(lanes, 2)
            return _svec(n_vec)
        return f

    def _nchunk(k, carry):
        start = wid * 320 + k * 64
        pltpu.sync_copy(accf_hbm.at[pl.ds(start, 64)], a0)
        _do_rows(64, a0, ob, _svec_nodes(start))
        pltpu.sync_copy(ob, outf_hbm.at[pl.ds(start, 64)])
        return carry
    lax.fori_loop(0, 5, _nchunk, 0)

    ubase = wid * 32
    pltpu.sync_copy(uid_hbm.at[pl.ds(ubase, 32)], uidv)
    for g in range(2):
        uidv[pl.ds(g * 16, 16)] = uidv[pl.ds(g * 16, 16)] + NUM_ITEMS
    pltpu.async_copy(accf_hbm.at[uidv], u0, sem).wait()

    def _svec_users(g):
        half = uidv[pl.ds((g // 4) * 16, 16)]
        idx = ((g % 4) * 4) + lax.shift_right_logical(lanes, 2)
        n_vec = half.at[idx].get(mode="promise_in_bounds")
        return _svec(n_vec)

    _do_rows(32, u0, uob, _svec_users)
    pltpu.sync_copy(uob, uout_hbm.at[pl.ds(ubase, 32)])


def _run_sc_final(accf, accw0, accw1, uid_pad, bias):
    mesh = plsc.VectorSubcoreMesh(core_axis_name="c", subcore_axis_name="s")
    fn = pl.kernel(
        _sc_final, mesh=mesh,
        out_type=[
            jax.ShapeDtypeStruct((NROWS, OUT), jnp.float32),
            jax.ShapeDtypeStruct((U_PAD, OUT), jnp.float32),
        ],
        scratch_types=[
            pltpu.VMEM((64, HO), jnp.float32),
            pltpu.VMEM((64, OUT), jnp.float32),
            pltpu.VMEM((WROWS, HO), jnp.float32),
            pltpu.VMEM((WROWS, HO), jnp.float32),
            pltpu.VMEM((32,), jnp.int32),
            pltpu.VMEM((32, HO), jnp.float32),
            pltpu.VMEM((32, OUT), jnp.float32),
            pltpu.VMEM((IN_NODE,), jnp.float32),
            pltpu.SemaphoreType.DMA,
        ],
        compiler_params=pltpu.CompilerParams(needs_layout_passes=False),
    )
    return fn(accf, accw0, accw1, uid_pad, bias)


def kernel(users_ids, users_features, items_features, edge_index, edge_reward,
           W_node, W_edge, attn_l, attn_r, attn_e, bias):
    x = jnp.concatenate([items_features, users_features], axis=0)
    fl = attn_l.reshape(1, HO)
    fr = attn_r.reshape(1, HO)
    fe = attn_e.reshape(1, HO)

    h, elr, crow = _tc_project(x, W_node, fl, fr, fe, W_edge)

    elr_flat = jnp.pad(elr, ((0, 1), (0, 0))).reshape(-1)
    c16 = crow[0, :16]

    src = edge_index[0]
    dst = edge_index[1]
    pad = E_PAD - E
    srcp = jnp.concatenate([src, jnp.zeros((pad,), src.dtype)])
    dstp = jnp.concatenate([dst, jnp.full((pad,), N, dst.dtype)])
    rwdp = jnp.concatenate([edge_reward, jnp.zeros((pad,), edge_reward.dtype)])

    wflat, accw0, accw1 = _run_sc_weights(elr_flat, srcp, dstp, rwdp, c16)
    accf = _run_sc_edges(h, srcp, dstp, wflat)

    uid_pad = jnp.concatenate(
        [users_ids, jnp.zeros((U_PAD - NUM_USERS,), users_ids.dtype)])
    outf, uout = _run_sc_final(accf, accw0, accw1, uid_pad, bias)

    return (uout[:NUM_USERS], outf[:NUM_ITEMS])
